# baseline (speedup 1.0000x reference)
import jax
import jax.numpy as jnp
from jax import lax
from jax.experimental import pallas as pl
from jax.experimental.pallas import tpu as pltpu
from jax.experimental.pallas import tpu_sc as plsc

VOCAB_SIZES = [99991, 100003, 100019, 100043, 100049, 100057, 100069, 100103]
H = len(VOCAB_SIZES)
D = 64
B, S = 4, 2048
N = B * S * H

_off = []
_acc = 0
for _v in VOCAB_SIZES:
    _off.append(_acc)
    _acc += _v
OFF16 = tuple(_off * 2)

NC, NS, L = 2, 16, 16
NW = NC * NS
CHUNK = 128
PER_W = N // NW
NCHUNK = PER_W // CHUNK


def _body(ids_hbm, table_hbm, off_hbm, out_hbm, idx_v, rows_v, off_v, sem):
    wid = lax.axis_index("s") * NC + lax.axis_index("c")
    row0 = wid * NCHUNK

    pltpu.sync_copy(ids_hbm.at[pl.ds(row0, NCHUNK)], idx_v)

    pltpu.sync_copy(off_hbm, off_v)
    off = off_v[...]

    def add_row(r, _):
        for c in range(CHUNK // L):
            sl = pl.ds(c * L, L)
            idx_v[r, sl] = idx_v[r, sl] + off
        return 0

    lax.fori_loop(0, NCHUNK, add_row, 0)

    def do_chunk(j, _):
        pltpu.async_copy(table_hbm.at[idx_v.at[j]], rows_v, sem).wait()
        pltpu.sync_copy(rows_v, out_hbm.at[pl.ds((row0 + j) * CHUNK, CHUNK)])
        return 0

    lax.fori_loop(0, NCHUNK, do_chunk, 0)


@jax.jit
def kernel(input_ids, table):
    ids2d = input_ids.reshape(N // CHUNK, CHUNK)
    off16 = jnp.asarray(OFF16, dtype=jnp.int32)
    mesh = plsc.VectorSubcoreMesh(core_axis_name="c", subcore_axis_name="s")
    out = pl.kernel(
        _body,
        mesh=mesh,
        out_type=jax.ShapeDtypeStruct((N, D), jnp.float32),
        compiler_params=pltpu.CompilerParams(use_tc_tiling_on_sc=False),
        scratch_types=[
            pltpu.VMEM((NCHUNK, CHUNK), jnp.int32),
            pltpu.VMEM((CHUNK, D), jnp.float32),
            pltpu.VMEM((L,), jnp.int32),
            pltpu.SemaphoreType.DMA,
        ],
    )(ids2d, table, off16)
    return out.reshape(B, S, H, D)


# speedup vs baseline: 1.0196x; 1.0196x over previous
import jax
import jax.numpy as jnp
from jax import lax
from jax.experimental import pallas as pl
from jax.experimental.pallas import tpu as pltpu
from jax.experimental.pallas import tpu_sc as plsc

VOCAB_SIZES = [99991, 100003, 100019, 100043, 100049, 100057, 100069, 100103]
H = len(VOCAB_SIZES)
D = 64
B, S = 4, 2048
N = B * S * H

_off = []
_acc = 0
for _v in VOCAB_SIZES:
    _off.append(_acc)
    _acc += _v
OFF16 = tuple(_off * 2)

NC, NS, L = 2, 16, 16
NW = NC * NS
CHUNK = 128
PER_W = N // NW
NCHUNK = PER_W // CHUNK


NBUF = 4
DEPTH = 2


def _body(ids_hbm, table_hbm, off_hbm, out_hbm, idx_v, bufs_v, off_v, *sems):
    gsems = sems[:NBUF]
    wsems = sems[NBUF:]
    wid = lax.axis_index("s") * NC + lax.axis_index("c")
    row0 = wid * NCHUNK

    pltpu.sync_copy(ids_hbm.at[pl.ds(row0, NCHUNK)], idx_v)

    pltpu.sync_copy(off_hbm, off_v)
    off = off_v[...]

    def add_row(r):
        for c in range(CHUNK // L):
            sl = pl.ds(c * L, L)
            idx_v[r, sl] = idx_v[r, sl] + off

    g = [None] * NCHUNK
    w = [None] * NCHUNK

    def retire(j):
        g[j].wait()
        w[j] = pltpu.async_copy(
            bufs_v.at[j % NBUF],
            out_hbm.at[pl.ds((row0 + j) * CHUNK, CHUNK)],
            wsems[j % NBUF],
        )

    for j in range(NCHUNK):
        b = j % NBUF
        if j >= NBUF:
            w[j - NBUF].wait()
        add_row(j)
        g[j] = pltpu.async_copy(table_hbm.at[idx_v.at[j]], bufs_v.at[b], gsems[b])
        if j >= DEPTH:
            retire(j - DEPTH)
    for j in range(NCHUNK - DEPTH, NCHUNK):
        retire(j)
    for j in range(NCHUNK - NBUF, NCHUNK):
        w[j].wait()


@jax.jit
def kernel(input_ids, table):
    ids2d = input_ids.reshape(N // CHUNK, CHUNK)
    off16 = jnp.asarray(OFF16, dtype=jnp.int32)
    mesh = plsc.VectorSubcoreMesh(core_axis_name="c", subcore_axis_name="s")
    out = pl.kernel(
        _body,
        mesh=mesh,
        out_type=jax.ShapeDtypeStruct((N, D), jnp.float32),
        compiler_params=pltpu.CompilerParams(use_tc_tiling_on_sc=False),
        scratch_types=[
            pltpu.VMEM((NCHUNK, CHUNK), jnp.int32),
            pltpu.VMEM((NBUF, CHUNK, D), jnp.float32),
            pltpu.VMEM((L,), jnp.int32),
        ]
        + [pltpu.SemaphoreType.DMA] * (2 * NBUF),
    )(ids2d, table, off16)
    return out.reshape(B, S, H, D)
